# baseline (speedup 1.0000x reference)
import numpy as np
import jax
import jax.numpy as jnp
from jax.experimental import pallas as pl
from jax.experimental.pallas import tpu as pltpu

BT = 256

C0, H0, W0 = 3, 32, 32
C1, K1, P1 = 10, 3, 3
H1, W1 = 30, 30
HP, WP = 10, 10
C2, K2, P2 = 5, 3, 4
H2, W2 = 8, 8
HQ, WQ = 2, 2
NF = C2 * HQ * WQ
NOUT = 2

_M1 = np.full((C1 * 96, C0 * 160), 270, np.int32)
for _co in range(C1):
    for _r in range(3):
        for _wo in range(W1):
            for _ci in range(C0):
                for _i in range(K1):
                    for _j in range(K1):
                        _M1[_co * 96 + _r * 32 + (_wo % 3) * 10 + _wo // 3,
                            _ci * 160 + (_r + _i) * 32 + (_wo + _j)] = (
                            _co * 27 + _ci * 9 + _i * 3 + _j)

_M2 = np.full((C2 * 64, HP * 160), 450, np.int32)
for _co in range(C2):
    for _h2 in range(H2):
        for _w2 in range(W2):
            for _ci in range(C1):
                for _i in range(K2):
                    for _j in range(K2):
                        _M2[_co * 64 + _h2 * 8 + (_w2 % 4) * 2 + _w2 // 4,
                            (_h2 + _i) * 160 + _ci * 16 + (_w2 + _j)] = (
                            _co * 90 + _ci * 9 + _i * 3 + _j)


def _body(x_ref, a1_ref, a2_ref, wlm_ref, o_ref,
          o1a_ref, o1b_ref, hsa_ref, hsb_ref, p1_ref, o2_ref, feat_ref):
    f32 = jnp.float32
    p1_ref[...] = jnp.zeros(p1_ref.shape, jnp.bfloat16)

    a1 = a1_ref[...]
    for ph in range(HP):
        o1_ref = o1a_ref if ph % 2 == 0 else o1b_ref
        hs_ref = hsa_ref if ph % 2 == 0 else hsb_ref
        s = x_ref[:, 3 * ph:3 * ph + 5, :, :].reshape(C0 * 160, BT)
        o1_ref[...] = jnp.dot(a1, s, preferred_element_type=f32).reshape(
            C1, 3, 32, BT)
        hs_ref[...] = jnp.maximum(jnp.maximum(o1_ref[:, 0], o1_ref[:, 1]),
                                  o1_ref[:, 2])
        pooled = jnp.maximum(
            jnp.maximum(hs_ref[:, 0:WP, :], hs_ref[:, WP:2 * WP, :]),
            hs_ref[:, 2 * WP:3 * WP, :]).astype(jnp.bfloat16)
        for ci in range(C1):
            p1_ref[pl.ds(ph * 160 + ci * 16, WP), :] = pooled[ci]

    o2_ref[...] = jnp.dot(a2_ref[...], p1_ref[...],
                          preferred_element_type=f32).reshape(C2, H2, W2, BT)
    feat_ref[pl.ds(NF, 4), :] = (
        jax.lax.broadcasted_iota(jnp.int32, (4, BT), 0) == 0).astype(f32)
    for qh in range(HQ):
        hm = jnp.max(o2_ref[:, 4 * qh:4 * qh + 4], axis=1)
        p2 = jnp.maximum(jnp.maximum(hm[:, 0:2], hm[:, 2:4]),
                         jnp.maximum(hm[:, 4:6], hm[:, 6:8]))
        for c in range(C2):
            feat_ref[pl.ds(c * 4 + qh * 2, WQ), :] = p2[c]

    res = jnp.dot(wlm_ref[...], feat_ref[...], preferred_element_type=f32)
    o_ref[...] = res[0:NOUT]


@jax.jit
def kernel(x_nchw, w1, b1, w2, b2, wl, bl):
    B = x_nchw.shape[0]
    Bp = ((B + BT - 1) // BT) * BT

    a1 = jnp.concatenate([w1, jnp.zeros(1, jnp.float32)])[_M1].astype(
        jnp.bfloat16)
    a2 = jnp.concatenate([w2, jnp.zeros(1, jnp.float32)])[_M2].astype(
        jnp.bfloat16)

    s2 = w2.reshape(C2, C1, 9).sum(-1)
    b2eff = b2 + s2 @ b1
    bleff = bl + wl.reshape(NOUT, NF) @ jnp.repeat(b2eff, HQ * WQ)
    wlm = jnp.zeros((8, 24), jnp.float32)
    wlm = wlm.at[:NOUT, :NF].set(wl.reshape(NOUT, NF))
    wlm = wlm.at[:NOUT, NF].set(bleff)

    x = jnp.transpose(x_nchw.astype(jnp.bfloat16), (1, 2, 3, 0))
    x = jnp.pad(x, ((0, 0), (0, 0), (0, 0), (0, Bp - B)))

    out = pl.pallas_call(
        _body,
        out_shape=jax.ShapeDtypeStruct((NOUT, Bp), jnp.float32),
        grid=(Bp // BT,),
        in_specs=[
            pl.BlockSpec((C0, H0, W0, BT), lambda g: (0, 0, 0, g)),
            pl.BlockSpec((C1 * 96, C0 * 160), lambda g: (0, 0)),
            pl.BlockSpec((C2 * 64, HP * 160), lambda g: (0, 0)),
            pl.BlockSpec((8, 24), lambda g: (0, 0)),
        ],
        out_specs=pl.BlockSpec((NOUT, BT), lambda g: (0, g)),
        scratch_shapes=[
            pltpu.VMEM((C1, 3, 32, BT), jnp.float32),
            pltpu.VMEM((C1, 3, 32, BT), jnp.float32),
            pltpu.VMEM((C1, 32, BT), jnp.float32),
            pltpu.VMEM((C1, 32, BT), jnp.float32),
            pltpu.VMEM((HP * 160, BT), jnp.bfloat16),
            pltpu.VMEM((C2, H2, W2, BT), jnp.float32),
            pltpu.VMEM((24, BT), jnp.float32),
        ],
        compiler_params=pltpu.CompilerParams(
            dimension_semantics=("parallel",)),
    )(x, a1, a2, wlm)
    return out[:, :B].T


# speedup vs baseline: 55.0072x; 55.0072x over previous
import jax
import jax.numpy as jnp
from jax.experimental import pallas as pl
from jax.experimental.pallas import tpu as pltpu

BT = 256

C0, H0, W0 = 3, 32, 32
C1, K1, P1 = 10, 3, 3
H1, W1 = 30, 30
HP, WP = 10, 10
C2, K2, P2 = 5, 3, 4
H2, W2 = 8, 8
HQ, WQ = 2, 2
NF = C2 * HQ * WQ
NOUT = 2

def _conv1_operator(w1):
    w1r = w1.reshape(C1, C0, K1, K1)
    w1p = jnp.pad(w1r, ((0, 0), (0, 0), (0, 0), (0, 30)))
    band = jnp.broadcast_to(w1p[:, :, :, None, :],
                            (C1, C0, K1, W1, 33)).reshape(C1, C0, K1, 990)
    band = band[:, :, :, :960].reshape(C1, C0, K1, W1, 32)
    rows = [jnp.pad(band, ((0, 0), (0, 0), (r, 2 - r), (0, 0), (0, 0)))
            for r in range(3)]
    t = jnp.stack(rows, axis=1)
    t = t.reshape(C1, 3, C0, 5, WP, 3, 32)
    t = t.transpose(0, 1, 5, 4, 2, 3, 6)
    t = t.reshape(C1, 3, 30, C0, 5, 32)
    t = jnp.pad(t, ((0, 0), (0, 0), (0, 2), (0, 0), (0, 0), (0, 0)))
    return t.reshape(C1 * 96, C0 * 160)


def _conv2_operator(w2):
    w2r = w2.reshape(C2, C1, K2, K2)
    w2p = jnp.pad(w2r, ((0, 0), (0, 0), (0, 0), (0, 14)))
    band = jnp.broadcast_to(w2p[:, :, :, None, :],
                            (C2, C1, K2, H2, 17)).reshape(C2, C1, K2, 136)
    band = band[:, :, :, :128].reshape(C2, C1, K2, H2, 16)
    rows = [jnp.pad(band, ((0, 0), (0, 0), (h, 7 - h), (0, 0), (0, 0)))
            for h in range(H2)]
    t = jnp.stack(rows, axis=1)
    t = t.reshape(C2, H2, C1, HP, WQ, 4, 16)
    t = t.transpose(0, 1, 5, 4, 3, 2, 6)
    t = t.reshape(C2, H2, 8, HP, C1, 16)
    return t.reshape(C2 * 64, HP * 160)


def _body(x_ref, a1_ref, a2_ref, wlm_ref, o_ref,
          o1a_ref, o1b_ref, hsa_ref, hsb_ref, p1_ref, o2_ref, feat_ref):
    f32 = jnp.float32
    p1_ref[...] = jnp.zeros(p1_ref.shape, jnp.bfloat16)

    a1 = a1_ref[...]
    for ph in range(HP):
        o1_ref = o1a_ref if ph % 2 == 0 else o1b_ref
        hs_ref = hsa_ref if ph % 2 == 0 else hsb_ref
        s = x_ref[:, 3 * ph:3 * ph + 5, :, :].reshape(C0 * 160, BT)
        o1_ref[...] = jnp.dot(a1, s, preferred_element_type=f32).reshape(
            C1, 3, 32, BT)
        hs_ref[...] = jnp.maximum(jnp.maximum(o1_ref[:, 0], o1_ref[:, 1]),
                                  o1_ref[:, 2])
        pooled = jnp.maximum(
            jnp.maximum(hs_ref[:, 0:WP, :], hs_ref[:, WP:2 * WP, :]),
            hs_ref[:, 2 * WP:3 * WP, :]).astype(jnp.bfloat16)
        for ci in range(C1):
            p1_ref[pl.ds(ph * 160 + ci * 16, WP), :] = pooled[ci]

    o2_ref[...] = jnp.dot(a2_ref[...], p1_ref[...],
                          preferred_element_type=f32).reshape(C2, H2, W2, BT)
    feat_ref[pl.ds(NF, 4), :] = (
        jax.lax.broadcasted_iota(jnp.int32, (4, BT), 0) == 0).astype(f32)
    for qh in range(HQ):
        hm = jnp.max(o2_ref[:, 4 * qh:4 * qh + 4], axis=1)
        p2 = jnp.maximum(jnp.maximum(hm[:, 0:2], hm[:, 2:4]),
                         jnp.maximum(hm[:, 4:6], hm[:, 6:8]))
        for c in range(C2):
            feat_ref[pl.ds(c * 4 + qh * 2, WQ), :] = p2[c]

    res = jnp.dot(wlm_ref[...], feat_ref[...], preferred_element_type=f32)
    o_ref[...] = res[0:NOUT]


@jax.jit
def kernel(x_nchw, w1, b1, w2, b2, wl, bl):
    B = x_nchw.shape[0]
    Bp = ((B + BT - 1) // BT) * BT

    a1 = _conv1_operator(w1).astype(jnp.bfloat16)
    a2 = _conv2_operator(w2).astype(jnp.bfloat16)

    s2 = w2.reshape(C2, C1, 9).sum(-1)
    b2eff = b2 + s2 @ b1
    bleff = bl + wl.reshape(NOUT, NF) @ jnp.repeat(b2eff, HQ * WQ)
    wlm = jnp.zeros((8, 24), jnp.float32)
    wlm = wlm.at[:NOUT, :NF].set(wl.reshape(NOUT, NF))
    wlm = wlm.at[:NOUT, NF].set(bleff)

    x = jnp.transpose(x_nchw.astype(jnp.bfloat16), (1, 2, 3, 0))
    x = jnp.pad(x, ((0, 0), (0, 0), (0, 0), (0, Bp - B)))

    out = pl.pallas_call(
        _body,
        out_shape=jax.ShapeDtypeStruct((NOUT, Bp), jnp.float32),
        grid=(Bp // BT,),
        in_specs=[
            pl.BlockSpec((C0, H0, W0, BT), lambda g: (0, 0, 0, g)),
            pl.BlockSpec((C1 * 96, C0 * 160), lambda g: (0, 0)),
            pl.BlockSpec((C2 * 64, HP * 160), lambda g: (0, 0)),
            pl.BlockSpec((8, 24), lambda g: (0, 0)),
        ],
        out_specs=pl.BlockSpec((NOUT, BT), lambda g: (0, g)),
        scratch_shapes=[
            pltpu.VMEM((C1, 3, 32, BT), jnp.float32),
            pltpu.VMEM((C1, 3, 32, BT), jnp.float32),
            pltpu.VMEM((C1, 32, BT), jnp.float32),
            pltpu.VMEM((C1, 32, BT), jnp.float32),
            pltpu.VMEM((HP * 160, BT), jnp.bfloat16),
            pltpu.VMEM((C2, H2, W2, BT), jnp.float32),
            pltpu.VMEM((24, BT), jnp.float32),
        ],
        compiler_params=pltpu.CompilerParams(
            dimension_semantics=("parallel",)),
    )(x, a1, a2, wlm)
    return out[:, :B].T
